# speedup vs baseline: 1.5352x; 1.5352x over previous
import functools

import jax
import jax.numpy as jnp
from jax import lax
from jax.experimental import pallas as pl
from jax.experimental.pallas import tpu as pltpu
from jax.experimental.pallas import tpu_sc as plsc

NC = 2
NS = 16
NW = NC * NS

B = 4 * 2048
D = 1024
CHUNK = 32
B_PER_W = B // NW
NCH = B_PER_W // CHUNK

_mesh = plsc.VectorSubcoreMesh(core_axis_name="c", subcore_axis_name="s")


@functools.partial(
    pl.kernel,
    out_type=jax.ShapeDtypeStruct((B, D), jnp.float32),
    mesh=_mesh,
    scratch_types=[
        pltpu.VMEM((NCH, CHUNK), jnp.int32),
        pltpu.VMEM((CHUNK, D), jnp.float32),
        pltpu.VMEM((CHUNK, D), jnp.float32),
        pltpu.SemaphoreType.DMA,
        pltpu.SemaphoreType.DMA,
    ],
)
def _gather_kernel(idx_hbm, table_hbm, out_hbm, idx_v, buf_a, buf_b, sem_a, sem_b):
    wid = lax.axis_index("s") * NC + lax.axis_index("c")
    base = wid * B_PER_W
    pltpu.sync_copy(idx_hbm.at[pl.ds(wid * NCH, NCH)], idx_v)

    bufs = (buf_a, buf_b)
    sems = (sem_a, sem_b)

    def gather(j):
        return pltpu.async_copy(table_hbm.at[idx_v.at[j]], bufs[j % 2], sems[j % 2])

    cp = gather(0)
    for j in range(1, NCH + 1):
        nxt = gather(j) if j < NCH else None
        cp.wait()
        pltpu.sync_copy(
            bufs[(j - 1) % 2], out_hbm.at[pl.ds(base + (j - 1) * CHUNK, CHUNK)]
        )
        cp = nxt


def kernel(x, table):
    idx = x.reshape(B // CHUNK, CHUNK).astype(jnp.int32)
    out = _gather_kernel(idx, table)
    return out.reshape(x.shape + (D,))


# speedup vs baseline: 1.5382x; 1.0019x over previous
import functools

import jax
import jax.numpy as jnp
from jax import lax
from jax.experimental import pallas as pl
from jax.experimental.pallas import tpu as pltpu
from jax.experimental.pallas import tpu_sc as plsc

NC = 2
NS = 16
NW = NC * NS

B = 4 * 2048
D = 1024
CHUNK = 16
NBUF = 4
B_PER_W = B // NW
NCH = B_PER_W // CHUNK

_mesh = plsc.VectorSubcoreMesh(core_axis_name="c", subcore_axis_name="s")


@functools.partial(
    pl.kernel,
    out_type=jax.ShapeDtypeStruct((B, D), jnp.float32),
    mesh=_mesh,
    scratch_types=[
        pltpu.VMEM((NCH, CHUNK), jnp.int32),
        [pltpu.VMEM((CHUNK, D), jnp.float32) for _ in range(NBUF)],
        [pltpu.SemaphoreType.DMA for _ in range(NBUF)],
        [pltpu.SemaphoreType.DMA for _ in range(NBUF)],
    ],
)
def _gather_kernel(idx_hbm, table_hbm, out_hbm, idx_v, bufs, gsems, wsems):
    wid = lax.axis_index("s") * NC + lax.axis_index("c")
    base = wid * B_PER_W
    pltpu.sync_copy(idx_hbm.at[pl.ds(wid * NCH, NCH)], idx_v)

    gd = [None] * NCH
    wd = [None] * NBUF
    LAG = NBUF - 2

    def write(k):
        b = k % NBUF
        gd[k].wait()
        wd[b] = pltpu.async_copy(
            bufs[b], out_hbm.at[pl.ds(base + k * CHUNK, CHUNK)], wsems[b]
        )

    for j in range(NCH):
        b = j % NBUF
        if wd[b] is not None:
            wd[b].wait()
        gd[j] = pltpu.async_copy(table_hbm.at[idx_v.at[j]], bufs[b], gsems[b])
        if j >= LAG:
            write(j - LAG)
    for k in range(NCH - LAG, NCH):
        write(k)
    for b in range(NBUF):
        if wd[b] is not None:
            wd[b].wait()


def kernel(x, table):
    idx = x.reshape(B // CHUNK, CHUNK).astype(jnp.int32)
    out = _gather_kernel(idx, table)
    return out.reshape(x.shape + (D,))


# speedup vs baseline: 1.5387x; 1.0003x over previous
import functools

import jax
import jax.numpy as jnp
from jax import lax
from jax.experimental import pallas as pl
from jax.experimental.pallas import tpu as pltpu
from jax.experimental.pallas import tpu_sc as plsc

NC = 2
NS = 16
NW = NC * NS

R = 4
C = 2048
B = R * C
D = 1024
CHUNK = 16
NBUF = 4
B_PER_W = B // NW
NCH = B_PER_W // CHUNK
W_PER_ROW = C // B_PER_W

_mesh = plsc.VectorSubcoreMesh(core_axis_name="c", subcore_axis_name="s")


@functools.partial(
    pl.kernel,
    out_type=jax.ShapeDtypeStruct((B, D), jnp.float32),
    mesh=_mesh,
    scratch_types=[
        pltpu.VMEM((B_PER_W,), jnp.int32),
        [pltpu.VMEM((CHUNK, D), jnp.float32) for _ in range(NBUF)],
        [pltpu.SemaphoreType.DMA for _ in range(NBUF)],
        [pltpu.SemaphoreType.DMA for _ in range(NBUF)],
    ],
)
def _gather_kernel(idx_hbm, table_hbm, out_hbm, idx_v, bufs, gsems, wsems):
    wid = lax.axis_index("s") * NC + lax.axis_index("c")
    base = wid * B_PER_W
    row = wid // W_PER_ROW
    col = (wid % W_PER_ROW) * B_PER_W
    pltpu.sync_copy(idx_hbm.at[row, pl.ds(col, B_PER_W)], idx_v)

    gd = [None] * NCH
    wd = [None] * NBUF
    LAG = NBUF - 2

    def write(k):
        b = k % NBUF
        gd[k].wait()
        wd[b] = pltpu.async_copy(
            bufs[b], out_hbm.at[pl.ds(base + k * CHUNK, CHUNK)], wsems[b]
        )

    for j in range(NCH):
        b = j % NBUF
        if wd[b] is not None:
            wd[b].wait()
        gd[j] = pltpu.async_copy(
            table_hbm.at[idx_v.at[pl.ds(j * CHUNK, CHUNK)]], bufs[b], gsems[b]
        )
        if j >= LAG:
            write(j - LAG)
    for k in range(NCH - LAG, NCH):
        write(k)
    for b in range(NBUF):
        if wd[b] is not None:
            wd[b].wait()


def kernel(x, table):
    out = _gather_kernel(x.astype(jnp.int32), table)
    return out.reshape(x.shape + (D,))
